# baseline (speedup 1.0000x reference)
import functools

import jax
import jax.numpy as jnp
from jax import lax
from jax.experimental import pallas as pl
from jax.experimental.pallas import tpu as pltpu
from jax.experimental.pallas import tpu_sc as plsc

N = 10000
E = 320000
D_IN = 128
D_H = 128
D_OUT = 64
B = 64

NC = 2
NS = 16
NW = NC * NS
NPAD = 10240
ROWS_PER_TILE = NPAD // NS
E_PER_TILE = E // NW
CH = 80
NCH = E_PER_TILE // CH

_SELU_ALPHA = 1.6732632423543772
_SELU_SCALE = 1.0507009873554805

RB = 10240
GRID = NPAD // RB


def _mesh():
    return plsc.VectorSubcoreMesh(core_axis_name="c", subcore_axis_name="s")


def _sc_degree(srcf, dst3d):
    @functools.partial(
        pl.kernel,
        out_type=[
            jax.ShapeDtypeStruct((NC, NPAD), jnp.float32),
            jax.ShapeDtypeStruct((2, E), jnp.int32),
        ],
        mesh=_mesh(),
        compiler_params=pltpu.CompilerParams(use_tc_tiling_on_sc=False),
        scratch_types=[
            pltpu.VMEM_SHARED((NPAD,), jnp.float32),
            pltpu.VMEM((ROWS_PER_TILE,), jnp.float32),
            pltpu.VMEM((CH,), jnp.float32),
            pltpu.VMEM((NCH_F, CH), jnp.int32),
            pltpu.VMEM((E_PER_TILE_F,), jnp.int32),
            pltpu.VMEM((E_PER_TILE_F,), jnp.int32),
            pltpu.SemaphoreType.DMA,
            pltpu.SemaphoreType.DMA,
        ],
    )
    def k(src_hbm, dst_hbm, out_hbm, src2_hbm, deg_sh, fill_v, ones_v,
          didx2d, sidx_v, todd_v, psem, sem):
        c = lax.axis_index("c")
        s = lax.axis_index("s")
        ebase = pl.multiple_of(s * E_PER_TILE_F, 8)
        pltpu.async_copy(dst_hbm.at[s], didx2d, psem)
        pltpu.async_copy(src_hbm.at[pl.ds(ebase, E_PER_TILE_F)], sidx_v, psem)
        f = jnp.where(c == 0, jnp.float32(1.0), jnp.float32(0.0))
        fvec = jnp.full((16,), 1.0, jnp.float32) * f
        for t in range(ROWS_PER_TILE // 16):
            fill_v[pl.ds(t * 16, 16)] = fvec
        one16 = jnp.full((16,), 1.0, jnp.float32)
        for t in range(CH // 16):
            ones_v[pl.ds(t * 16, 16)] = one16
        pltpu.sync_copy(fill_v, deg_sh.at[pl.ds(s * ROWS_PER_TILE, ROWS_PER_TILE)])
        pltpu.make_async_copy(dst_hbm.at[s], didx2d, psem).wait()
        pltpu.make_async_copy(src_hbm.at[pl.ds(ebase, E_PER_TILE_F)], sidx_v, psem).wait()
        plsc.subcore_barrier()
        cbase = c * NCH

        GF = 5

        def body(gr, carry):
            for j in range(GF):
                pltpu.async_copy(
                    ones_v, deg_sh.at[didx2d.at[cbase + gr * GF + j]], sem, add=True)
            for j in range(GF):
                pltpu.make_async_copy(
                    ones_v, deg_sh.at[didx2d.at[0]], sem).wait()
            return carry

        lax.fori_loop(0, NCH // GF, body, 0)

        one16i = jnp.full((16,), 1, jnp.int32)

        def tbody(i, carry):
            off = pl.multiple_of(i * CH, 8)
            for q in range(CH // 16):
                xv = sidx_v[pl.ds(off + q * 16, 16)]
                yv = xv + xv
                sidx_v[pl.ds(off + q * 16, 16)] = yv
                todd_v[pl.ds(off + q * 16, 16)] = yv + one16i
            return carry

        lax.fori_loop(0, NCH_F, tbody, 0)
        pltpu.sync_copy(sidx_v, src2_hbm.at[0, pl.ds(ebase, E_PER_TILE_F)])
        pltpu.sync_copy(todd_v, src2_hbm.at[1, pl.ds(ebase, E_PER_TILE_F)])
        plsc.subcore_barrier()
        pltpu.sync_copy(
            deg_sh.at[pl.ds(s * ROWS_PER_TILE, ROWS_PER_TILE)],
            out_hbm.at[c, pl.ds(s * ROWS_PER_TILE, ROWS_PER_TILE)],
        )

    return k(srcf, dst3d)


DHALF = D_H // 2
E_PER_TILE_F = E // NS
NCH_F = E_PER_TILE_F // CH


def _sc_aggregate(hsv, src2, dst3d):
    @functools.partial(
        pl.kernel,
        out_type=jax.ShapeDtypeStruct((NPAD, D_H), jnp.float32),
        mesh=_mesh(),
        compiler_params=pltpu.CompilerParams(use_tc_tiling_on_sc=False),
        scratch_types=[
            pltpu.VMEM_SHARED((NPAD, DHALF), jnp.float32),
            pltpu.VMEM((E_PER_TILE_F,), jnp.int32),
            pltpu.VMEM((NCH_F, CH), jnp.int32),
            *[pltpu.VMEM((CH, DHALF), jnp.float32) for _ in range(8)],
            pltpu.SemaphoreType.DMA,
            *[pltpu.SemaphoreType.DMA for _ in range(8)],
            *[pltpu.SemaphoreType.DMA for _ in range(8)],
        ],
    )
    def k(hs_hbm, src_hbm, dst_hbm, out_hbm, acc_sh, sidx_all, didx2d,
          *bufs_and_sems):
        rows = bufs_and_sems[0:8]
        psem = bufs_and_sems[8]
        gsems = bufs_and_sems[9:17]
        ssems = bufs_and_sems[17:25]
        c = lax.axis_index("c")
        s = lax.axis_index("s")
        ebase = pl.multiple_of(s * E_PER_TILE_F, 8)
        pltpu.async_copy(src_hbm.at[c, pl.ds(ebase, E_PER_TILE_F)], sidx_all, psem)
        pltpu.async_copy(dst_hbm.at[s], didx2d, psem)
        rows0 = rows[0]
        zero16 = jnp.zeros((16,), jnp.float32)
        for r in range(CH):
            for q in range(DHALF // 16):
                rows0[r, pl.ds(q * 16, 16)] = zero16
        for t in range(ROWS_PER_TILE // CH):
            pltpu.sync_copy(rows0, acc_sh.at[pl.ds(s * ROWS_PER_TILE + t * CH, CH)])
        pltpu.make_async_copy(src_hbm.at[c, pl.ds(ebase, E_PER_TILE_F)], sidx_all, psem).wait()
        pltpu.make_async_copy(dst_hbm.at[s], didx2d, psem).wait()
        plsc.subcore_barrier()

        def gather(g, rows_v, gsem):
            pltpu.async_copy(
                hs_hbm.at[sidx_all.at[pl.ds(g * CH, CH)]], rows_v, gsem)

        def wait_gather(rows_v, gsem):
            pltpu.make_async_copy(
                hs_hbm.at[sidx_all.at[pl.ds(0, CH)]], rows_v, gsem).wait()

        def scatter(rows_v, g, ssem):
            pltpu.async_copy(rows_v, acc_sh.at[didx2d.at[g]], ssem, add=True)

        def wait_scatter(rows_v, ssem):
            pltpu.make_async_copy(rows_v, acc_sh.at[didx2d.at[0]], ssem).wait()

        H = 4
        S = 8
        for j in range(H):
            gather(j, rows[j], gsems[j])
        NLOOP = 30

        def body(k_, carry):
            a = S * k_

            for j in range(H):
                @pl.when(k_ > 0)
                def _(j=j):
                    wait_scatter(rows[H + j], ssems[H + j])
                gather(a + H + j, rows[H + j], gsems[H + j])
            for j in range(H):
                wait_gather(rows[j], gsems[j])
                scatter(rows[j], a + j, ssems[j])
            for j in range(H):
                wait_scatter(rows[j], ssems[j])
                gather(a + S + j, rows[j], gsems[j])
            for j in range(H):
                wait_gather(rows[H + j], gsems[H + j])
                scatter(rows[H + j], a + H + j, ssems[H + j])
            return carry

        lax.fori_loop(0, NLOOP, body, 0)
        a = S * NLOOP
        for j in range(H):
            wait_scatter(rows[H + j], ssems[H + j])
            gather(a + H + j, rows[H + j], gsems[H + j])
        for j in range(H):
            wait_gather(rows[j], gsems[j])
            scatter(rows[j], a + j, ssems[j])
        for j in range(2):
            wait_scatter(rows[j], ssems[j])
            gather(a + S + j, rows[j], gsems[j])
        for j in range(H):
            wait_gather(rows[H + j], gsems[H + j])
            scatter(rows[H + j], a + H + j, ssems[H + j])
        for j in range(2):
            wait_gather(rows[j], gsems[j])
            scatter(rows[j], a + S + j, ssems[j])
        for j in range(2):
            wait_scatter(rows[j], ssems[j])
        for j in range(2, H):
            wait_scatter(rows[j], ssems[j])
        for j in range(H):
            wait_scatter(rows[H + j], ssems[H + j])
        plsc.subcore_barrier()
        pltpu.sync_copy(
            acc_sh.at[pl.ds(s * ROWS_PER_TILE, ROWS_PER_TILE)],
            out_hbm.at[pl.ds(s * ROWS_PER_TILE, ROWS_PER_TILE), pl.ds(c * DHALF, DHALF)],
        )

    return k(hsv, src2, dst3d)


def _selu(t):
    return _SELU_SCALE * jnp.where(t > 0, t, _SELU_ALPHA * (jnp.exp(jnp.minimum(t, 0.0)) - 1.0))


def _tc_layer1(xp, W1, d0, d1):
    def body(x_ref, w_ref, d0_ref, d1_ref, hs_ref, dis_ref):
        dis = lax.rsqrt(d0_ref[...] + d1_ref[...])
        h = jnp.dot(x_ref[...], w_ref[...], preferred_element_type=jnp.float32)
        hs_ref[...] = h * dis
        dis_ref[...] = dis

    return pl.pallas_call(
        body,
        grid=(GRID,),
        in_specs=[
            pl.BlockSpec((RB, D_IN), lambda i: (i, 0)),
            pl.BlockSpec((D_IN, D_H), lambda i: (0, 0)),
            pl.BlockSpec((RB, 1), lambda i: (i, 0)),
            pl.BlockSpec((RB, 1), lambda i: (i, 0)),
        ],
        out_specs=[
            pl.BlockSpec((RB, D_H), lambda i: (i, 0)),
            pl.BlockSpec((RB, 1), lambda i: (i, 0)),
        ],
        out_shape=[
            jax.ShapeDtypeStruct((NPAD, D_H), jnp.float32),
            jax.ShapeDtypeStruct((NPAD, 1), jnp.float32),
        ],
    )(xp, W1, d0, d1)


def _tc_layer2(agg, hs1, dis, b1, W2):
    def body(agg_ref, hs_ref, dis_ref, b_ref, w_ref, out_ref):
        dis = dis_ref[...]
        t = dis * (agg_ref[...] + hs_ref[...]) + b_ref[...]
        act = _selu(t)
        h2 = jnp.dot(act, w_ref[...], preferred_element_type=jnp.float32)
        out_ref[...] = h2 * dis

    return pl.pallas_call(
        body,
        grid=(GRID,),
        in_specs=[
            pl.BlockSpec((RB, D_H), lambda i: (i, 0)),
            pl.BlockSpec((RB, D_H), lambda i: (i, 0)),
            pl.BlockSpec((RB, 1), lambda i: (i, 0)),
            pl.BlockSpec((1, D_H), lambda i: (0, 0)),
            pl.BlockSpec((D_H, D_H), lambda i: (0, 0)),
        ],
        out_specs=pl.BlockSpec((RB, D_H), lambda i: (i, 0)),
        out_shape=jax.ShapeDtypeStruct((NPAD, D_H), jnp.float32),
    )(agg, hs1, dis, b1, W2)


def _tc_head(agg, hs2, dis, b2, batchp, Wd, bd):
    def body(agg_ref, hs_ref, dis_ref, b_ref, bt_ref, wd_ref, bd_ref,
             out_ref, sums, cnts):
        i = pl.program_id(0)

        @pl.when(i == 0)
        def _init():
            sums[...] = jnp.zeros_like(sums)
            cnts[...] = jnp.zeros_like(cnts)

        dis = dis_ref[...]
        t = dis * (agg_ref[...] + hs_ref[...]) + b_ref[...]
        act = _selu(t)
        seg = bt_ref[...]
        oh = (seg == lax.broadcasted_iota(jnp.int32, (1, B), 1)).astype(jnp.float32)
        sums[...] += lax.dot_general(
            oh, act, (((0,), (0,)), ((), ())), preferred_element_type=jnp.float32)
        cnts[...] += lax.dot_general(
            oh, jnp.ones((RB, 1), jnp.float32), (((0,), (0,)), ((), ())),
            preferred_element_type=jnp.float32)

        @pl.when(i == GRID - 1)
        def _final():
            pooled = sums[...] / jnp.maximum(cnts[...], 1.0)
            logits = jnp.dot(pooled, wd_ref[...],
                             preferred_element_type=jnp.float32) + bd_ref[...]
            out_ref[...] = jax.nn.sigmoid(logits)

    return pl.pallas_call(
        body,
        grid=(GRID,),
        in_specs=[
            pl.BlockSpec((RB, D_H), lambda i: (i, 0)),
            pl.BlockSpec((RB, D_H), lambda i: (i, 0)),
            pl.BlockSpec((RB, 1), lambda i: (i, 0)),
            pl.BlockSpec((1, D_H), lambda i: (0, 0)),
            pl.BlockSpec((RB, 1), lambda i: (i, 0)),
            pl.BlockSpec((D_H, D_OUT), lambda i: (0, 0)),
            pl.BlockSpec((1, D_OUT), lambda i: (0, 0)),
        ],
        out_specs=pl.BlockSpec((B, D_OUT), lambda i: (0, 0)),
        out_shape=jax.ShapeDtypeStruct((B, D_OUT), jnp.float32),
        scratch_shapes=[
            pltpu.VMEM((B, D_H), jnp.float32),
            pltpu.VMEM((B, 1), jnp.float32),
        ],
    )(agg, hs2, dis, b2, batchp, Wd, bd)


def kernel(x, W1, b1, W2, b2, Wd, bd, edge_index, batch):
    dst3d = edge_index[1].reshape(NS, NCH_F, CH)
    degp, src2 = _sc_degree(edge_index[0], dst3d)
    d0 = degp[0][:, None]
    d1 = degp[1][:, None]
    xp = jnp.pad(x, ((0, NPAD - N), (0, 0)))
    hs1, dis = _tc_layer1(xp, W1, d0, d1)
    agg1 = _sc_aggregate(hs1.reshape(2 * NPAD, DHALF), src2, dst3d)
    hs2 = _tc_layer2(agg1, hs1, dis, b1[None, :], W2)
    agg2 = _sc_aggregate(hs2.reshape(2 * NPAD, DHALF), src2, dst3d)
    batchp = jnp.pad(batch, (0, NPAD - N), constant_values=B)[:, None]
    out = _tc_head(agg2, hs2, dis, b2[None, :], batchp, Wd, bd[None, :])
    return out


# speedup vs baseline: 1.0195x; 1.0195x over previous
import functools

import jax
import jax.numpy as jnp
from jax import lax
from jax.experimental import pallas as pl
from jax.experimental.pallas import tpu as pltpu
from jax.experimental.pallas import tpu_sc as plsc

N = 10000
E = 320000
D_IN = 128
D_H = 128
D_OUT = 64
B = 64

NC = 2
NS = 16
NW = NC * NS
NPAD = 10240
ROWS_PER_TILE = NPAD // NS
E_PER_TILE = E // NW
CH = 80
NCH = E_PER_TILE // CH

_SELU_ALPHA = 1.6732632423543772
_SELU_SCALE = 1.0507009873554805

RB = 5120
GRID = NPAD // RB


def _mesh():
    return plsc.VectorSubcoreMesh(core_axis_name="c", subcore_axis_name="s")


def _sc_degree(srcf, dst3d):
    @functools.partial(
        pl.kernel,
        out_type=[
            jax.ShapeDtypeStruct((NC, NPAD), jnp.float32),
            jax.ShapeDtypeStruct((2, E), jnp.int32),
        ],
        mesh=_mesh(),
        compiler_params=pltpu.CompilerParams(use_tc_tiling_on_sc=False),
        scratch_types=[
            pltpu.VMEM_SHARED((NPAD,), jnp.float32),
            pltpu.VMEM((ROWS_PER_TILE,), jnp.float32),
            pltpu.VMEM((CH,), jnp.float32),
            pltpu.VMEM((NCH_F, CH), jnp.int32),
            pltpu.VMEM((E_PER_TILE_F,), jnp.int32),
            pltpu.VMEM((E_PER_TILE_F,), jnp.int32),
            pltpu.SemaphoreType.DMA,
            pltpu.SemaphoreType.DMA,
        ],
    )
    def k(src_hbm, dst_hbm, out_hbm, src2_hbm, deg_sh, fill_v, ones_v,
          didx2d, sidx_v, todd_v, psem, sem):
        c = lax.axis_index("c")
        s = lax.axis_index("s")
        ebase = pl.multiple_of(s * E_PER_TILE_F, 8)
        pltpu.async_copy(dst_hbm.at[s], didx2d, psem)
        pltpu.async_copy(src_hbm.at[pl.ds(ebase, E_PER_TILE_F)], sidx_v, psem)
        f = jnp.where(c == 0, jnp.float32(1.0), jnp.float32(0.0))
        fvec = jnp.full((16,), 1.0, jnp.float32) * f
        for t in range(ROWS_PER_TILE // 16):
            fill_v[pl.ds(t * 16, 16)] = fvec
        one16 = jnp.full((16,), 1.0, jnp.float32)
        for t in range(CH // 16):
            ones_v[pl.ds(t * 16, 16)] = one16
        pltpu.sync_copy(fill_v, deg_sh.at[pl.ds(s * ROWS_PER_TILE, ROWS_PER_TILE)])
        pltpu.make_async_copy(dst_hbm.at[s], didx2d, psem).wait()
        pltpu.make_async_copy(src_hbm.at[pl.ds(ebase, E_PER_TILE_F)], sidx_v, psem).wait()
        plsc.subcore_barrier()
        cbase = c * NCH

        GF = 5

        def body(gr, carry):
            for j in range(GF):
                pltpu.async_copy(
                    ones_v, deg_sh.at[didx2d.at[cbase + gr * GF + j]], sem, add=True)
            for j in range(GF):
                pltpu.make_async_copy(
                    ones_v, deg_sh.at[didx2d.at[0]], sem).wait()
            return carry

        lax.fori_loop(0, NCH // GF, body, 0)

        one16i = jnp.full((16,), 1, jnp.int32)

        def tbody(i, carry):
            off = pl.multiple_of(i * CH, 8)
            for q in range(CH // 16):
                xv = sidx_v[pl.ds(off + q * 16, 16)]
                yv = xv + xv
                sidx_v[pl.ds(off + q * 16, 16)] = yv
                todd_v[pl.ds(off + q * 16, 16)] = yv + one16i
            return carry

        lax.fori_loop(0, NCH_F, tbody, 0)
        pltpu.sync_copy(sidx_v, src2_hbm.at[0, pl.ds(ebase, E_PER_TILE_F)])
        pltpu.sync_copy(todd_v, src2_hbm.at[1, pl.ds(ebase, E_PER_TILE_F)])
        plsc.subcore_barrier()
        pltpu.sync_copy(
            deg_sh.at[pl.ds(s * ROWS_PER_TILE, ROWS_PER_TILE)],
            out_hbm.at[c, pl.ds(s * ROWS_PER_TILE, ROWS_PER_TILE)],
        )

    return k(srcf, dst3d)


DHALF = D_H // 2
E_PER_TILE_F = E // NS
NCH_F = E_PER_TILE_F // CH


def _sc_aggregate(hsv, src2, dst3d):
    @functools.partial(
        pl.kernel,
        out_type=jax.ShapeDtypeStruct((NPAD, D_H), jnp.float32),
        mesh=_mesh(),
        compiler_params=pltpu.CompilerParams(use_tc_tiling_on_sc=False),
        scratch_types=[
            pltpu.VMEM_SHARED((NPAD, DHALF), jnp.float32),
            pltpu.VMEM((E_PER_TILE_F,), jnp.int32),
            pltpu.VMEM((NCH_F, CH), jnp.int32),
            *[pltpu.VMEM((CH, DHALF), jnp.float32) for _ in range(8)],
            pltpu.SemaphoreType.DMA,
            *[pltpu.SemaphoreType.DMA for _ in range(8)],
            *[pltpu.SemaphoreType.DMA for _ in range(8)],
        ],
    )
    def k(hs_hbm, src_hbm, dst_hbm, out_hbm, acc_sh, sidx_all, didx2d,
          *bufs_and_sems):
        rows = bufs_and_sems[0:8]
        psem = bufs_and_sems[8]
        gsems = bufs_and_sems[9:17]
        ssems = bufs_and_sems[17:25]
        c = lax.axis_index("c")
        s = lax.axis_index("s")
        ebase = pl.multiple_of(s * E_PER_TILE_F, 8)
        pltpu.async_copy(src_hbm.at[c, pl.ds(ebase, E_PER_TILE_F)], sidx_all, psem)
        pltpu.async_copy(dst_hbm.at[s], didx2d, psem)
        rows0 = rows[0]
        zero16 = jnp.zeros((16,), jnp.float32)
        for r in range(CH):
            for q in range(DHALF // 16):
                rows0[r, pl.ds(q * 16, 16)] = zero16
        for t in range(ROWS_PER_TILE // CH):
            pltpu.sync_copy(rows0, acc_sh.at[pl.ds(s * ROWS_PER_TILE + t * CH, CH)])
        pltpu.make_async_copy(src_hbm.at[c, pl.ds(ebase, E_PER_TILE_F)], sidx_all, psem).wait()
        pltpu.make_async_copy(dst_hbm.at[s], didx2d, psem).wait()
        plsc.subcore_barrier()

        def gather(g, rows_v, gsem):
            pltpu.async_copy(
                hs_hbm.at[sidx_all.at[pl.ds(g * CH, CH)]], rows_v, gsem)

        def wait_gather(rows_v, gsem):
            pltpu.make_async_copy(
                hs_hbm.at[sidx_all.at[pl.ds(0, CH)]], rows_v, gsem).wait()

        def scatter(rows_v, g, ssem):
            pltpu.async_copy(rows_v, acc_sh.at[didx2d.at[g]], ssem, add=True)

        def wait_scatter(rows_v, ssem):
            pltpu.make_async_copy(rows_v, acc_sh.at[didx2d.at[0]], ssem).wait()

        H = 4
        S = 8
        for j in range(H):
            gather(j, rows[j], gsems[j])
        NLOOP = 30

        def body(k_, carry):
            a = S * k_

            for j in range(H):
                @pl.when(k_ > 0)
                def _(j=j):
                    wait_scatter(rows[H + j], ssems[H + j])
                gather(a + H + j, rows[H + j], gsems[H + j])
            for j in range(H):
                wait_gather(rows[j], gsems[j])
                scatter(rows[j], a + j, ssems[j])
            for j in range(H):
                wait_scatter(rows[j], ssems[j])
                gather(a + S + j, rows[j], gsems[j])
            for j in range(H):
                wait_gather(rows[H + j], gsems[H + j])
                scatter(rows[H + j], a + H + j, ssems[H + j])
            return carry

        lax.fori_loop(0, NLOOP, body, 0)
        a = S * NLOOP
        for j in range(H):
            wait_scatter(rows[H + j], ssems[H + j])
            gather(a + H + j, rows[H + j], gsems[H + j])
        for j in range(H):
            wait_gather(rows[j], gsems[j])
            scatter(rows[j], a + j, ssems[j])
        for j in range(2):
            wait_scatter(rows[j], ssems[j])
            gather(a + S + j, rows[j], gsems[j])
        for j in range(H):
            wait_gather(rows[H + j], gsems[H + j])
            scatter(rows[H + j], a + H + j, ssems[H + j])
        for j in range(2):
            wait_gather(rows[j], gsems[j])
            scatter(rows[j], a + S + j, ssems[j])
        for j in range(2):
            wait_scatter(rows[j], ssems[j])
        for j in range(2, H):
            wait_scatter(rows[j], ssems[j])
        for j in range(H):
            wait_scatter(rows[H + j], ssems[H + j])
        plsc.subcore_barrier()
        pltpu.sync_copy(
            acc_sh.at[pl.ds(s * ROWS_PER_TILE, ROWS_PER_TILE)],
            out_hbm.at[pl.ds(s * ROWS_PER_TILE, ROWS_PER_TILE), pl.ds(c * DHALF, DHALF)],
        )

    return k(hsv, src2, dst3d)


def _selu(t):
    return _SELU_SCALE * jnp.where(t > 0, t, _SELU_ALPHA * (jnp.exp(jnp.minimum(t, 0.0)) - 1.0))


def _tc_layer1(xp, W1, d0, d1):
    def body(x_ref, w_ref, d0_ref, d1_ref, hs_ref, dis_ref):
        dis = lax.rsqrt(d0_ref[...] + d1_ref[...])
        h = jnp.dot(x_ref[...], w_ref[...], preferred_element_type=jnp.float32)
        hs_ref[...] = h * dis
        dis_ref[...] = dis

    return pl.pallas_call(
        body,
        grid=(GRID,),
        in_specs=[
            pl.BlockSpec((RB, D_IN), lambda i: (i, 0)),
            pl.BlockSpec((D_IN, D_H), lambda i: (0, 0)),
            pl.BlockSpec((RB, 1), lambda i: (i, 0)),
            pl.BlockSpec((RB, 1), lambda i: (i, 0)),
        ],
        out_specs=[
            pl.BlockSpec((RB, D_H), lambda i: (i, 0)),
            pl.BlockSpec((RB, 1), lambda i: (i, 0)),
        ],
        out_shape=[
            jax.ShapeDtypeStruct((NPAD, D_H), jnp.float32),
            jax.ShapeDtypeStruct((NPAD, 1), jnp.float32),
        ],
    )(xp, W1, d0, d1)


def _tc_layer2(agg, hs1, dis, b1, W2):
    def body(agg_ref, hs_ref, dis_ref, b_ref, w_ref, out_ref):
        dis = dis_ref[...]
        t = dis * (agg_ref[...] + hs_ref[...]) + b_ref[...]
        act = _selu(t)
        h2 = jnp.dot(act, w_ref[...], preferred_element_type=jnp.float32)
        out_ref[...] = h2 * dis

    return pl.pallas_call(
        body,
        grid=(GRID,),
        in_specs=[
            pl.BlockSpec((RB, D_H), lambda i: (i, 0)),
            pl.BlockSpec((RB, D_H), lambda i: (i, 0)),
            pl.BlockSpec((RB, 1), lambda i: (i, 0)),
            pl.BlockSpec((1, D_H), lambda i: (0, 0)),
            pl.BlockSpec((D_H, D_H), lambda i: (0, 0)),
        ],
        out_specs=pl.BlockSpec((RB, D_H), lambda i: (i, 0)),
        out_shape=jax.ShapeDtypeStruct((NPAD, D_H), jnp.float32),
    )(agg, hs1, dis, b1, W2)


def _tc_head(agg, hs2, dis, b2, batchp, Wd, bd):
    def body(agg_ref, hs_ref, dis_ref, b_ref, bt_ref, wd_ref, bd_ref,
             out_ref, sums, cnts):
        i = pl.program_id(0)

        @pl.when(i == 0)
        def _init():
            sums[...] = jnp.zeros_like(sums)
            cnts[...] = jnp.zeros_like(cnts)

        dis = dis_ref[...]
        t = dis * (agg_ref[...] + hs_ref[...]) + b_ref[...]
        act = _selu(t)
        seg = bt_ref[...]
        oh = (seg == lax.broadcasted_iota(jnp.int32, (1, B), 1)).astype(jnp.float32)
        sums[...] += lax.dot_general(
            oh, act, (((0,), (0,)), ((), ())), preferred_element_type=jnp.float32)
        cnts[...] += lax.dot_general(
            oh, jnp.ones((RB, 1), jnp.float32), (((0,), (0,)), ((), ())),
            preferred_element_type=jnp.float32)

        @pl.when(i == GRID - 1)
        def _final():
            pooled = sums[...] / jnp.maximum(cnts[...], 1.0)
            logits = jnp.dot(pooled, wd_ref[...],
                             preferred_element_type=jnp.float32) + bd_ref[...]
            out_ref[...] = jax.nn.sigmoid(logits)

    return pl.pallas_call(
        body,
        grid=(GRID,),
        in_specs=[
            pl.BlockSpec((RB, D_H), lambda i: (i, 0)),
            pl.BlockSpec((RB, D_H), lambda i: (i, 0)),
            pl.BlockSpec((RB, 1), lambda i: (i, 0)),
            pl.BlockSpec((1, D_H), lambda i: (0, 0)),
            pl.BlockSpec((RB, 1), lambda i: (i, 0)),
            pl.BlockSpec((D_H, D_OUT), lambda i: (0, 0)),
            pl.BlockSpec((1, D_OUT), lambda i: (0, 0)),
        ],
        out_specs=pl.BlockSpec((B, D_OUT), lambda i: (0, 0)),
        out_shape=jax.ShapeDtypeStruct((B, D_OUT), jnp.float32),
        scratch_shapes=[
            pltpu.VMEM((B, D_H), jnp.float32),
            pltpu.VMEM((B, 1), jnp.float32),
        ],
    )(agg, hs2, dis, b2, batchp, Wd, bd)


def kernel(x, W1, b1, W2, b2, Wd, bd, edge_index, batch):
    dst3d = edge_index[1].reshape(NS, NCH_F, CH)
    degp, src2 = _sc_degree(edge_index[0], dst3d)
    d0 = degp[0][:, None]
    d1 = degp[1][:, None]
    xp = jnp.pad(x, ((0, NPAD - N), (0, 0)))
    hs1, dis = _tc_layer1(xp, W1, d0, d1)
    agg1 = _sc_aggregate(hs1.reshape(2 * NPAD, DHALF), src2, dst3d)
    hs2 = _tc_layer2(agg1, hs1, dis, b1[None, :], W2)
    agg2 = _sc_aggregate(hs2.reshape(2 * NPAD, DHALF), src2, dst3d)
    batchp = jnp.pad(batch, (0, NPAD - N), constant_values=B)[:, None]
    out = _tc_head(agg2, hs2, dis, b2[None, :], batchp, Wd, bd[None, :])
    return out
